# baseline (speedup 1.0000x reference)
import functools

import jax
import jax.numpy as jnp
from jax import lax
from jax.experimental import pallas as pl
from jax.experimental.pallas import tpu as pltpu
from jax.experimental.pallas import tpu_sc as plsc

N_NODES = 10000
N_EDGES = 160000
D = 256
H = 128
RANK = 32
B = 4096

NC = 2
NS = 16
L = 16

EPT = N_EDGES // NS
C = 80
NCH = EPT // C
NBUF = 3
ZR = 400
NZCH = N_NODES // ZR
WR = 80
NWCH = N_NODES // WR

_mesh = plsc.VectorSubcoreMesh(core_axis_name="c", subcore_axis_name="s")


@functools.partial(
    pl.kernel,
    out_type=jax.ShapeDtypeStruct((NC, N_NODES, H), jnp.float32),
    mesh=_mesh,
    scratch_types=[
        pltpu.VMEM((NBUF, 2, C), jnp.int32),
        pltpu.VMEM((NCH, C), jnp.float32),
        pltpu.VMEM((NBUF, C, H), jnp.float32),
        pltpu.VMEM_SHARED((N_NODES, H), jnp.float32),
        pltpu.SemaphoreType.DMA((NBUF,)),
        pltpu.SemaphoreType.DMA((NBUF,)),
    ],
)
def _edge_agg(x2, edges, ws, zrows, agg, ec_v, w_v, rows_v, acc, gsem, ssem):
    c = lax.axis_index("c")
    s = lax.axis_index("s")

    def zcp(k, carry):
        q = s + k * NS
        pltpu.sync_copy(zrows, acc.at[pl.ds(q * ZR, ZR)])
        return carry

    lax.fori_loop(0, (NZCH - s + NS - 1) // NS, zcp, 0)

    pltpu.sync_copy(ws.at[s], w_v)
    plsc.subcore_barrier()

    def stage(j, b):
        pltpu.sync_copy(edges.at[c, s, j], ec_v.at[b])
        pltpu.async_copy(x2.at[ec_v.at[b, 0]], rows_v.at[b], gsem.at[b])

    def wait_gather(b):
        pltpu.make_async_copy(x2.at[pl.ds(0, C)], rows_v.at[b],
                              gsem.at[b]).wait()

    def wait_scatter(b):
        pltpu.make_async_copy(rows_v.at[b], acc.at[pl.ds(0, C)],
                              ssem.at[b]).wait()

    stage(0, 0)
    stage(1, 1)

    def chunk(j, carry):
        b = lax.rem(j, NBUF)
        b2 = lax.rem(j + 2, NBUF)

        @pl.when(j + 2 < NCH)
        def _():
            @pl.when(j >= 1)
            def _():
                wait_scatter(b2)
            stage(j + 2, b2)

        wait_gather(b)

        def scale16(e16, carry2):
            w16 = w_v[j, pl.ds(e16 * L, L)]
            for l in range(L):
                wsp = w16.at[jnp.full((L,), l, jnp.int32)].get(
                    mode="promise_in_bounds")
                e = e16 * L + l
                for k in range(H // L):
                    sl = pl.ds(k * L, L)
                    rows_v[b, e, sl] = rows_v[b, e, sl] * wsp
            return carry2

        lax.fori_loop(0, C // L, scale16, 0)
        pltpu.async_copy(rows_v.at[b], acc.at[ec_v.at[b, 1]], ssem.at[b],
                         add=True)
        return carry

    lax.fori_loop(0, NCH, chunk, 0)
    for b in range(NBUF):
        wait_scatter(b)
    plsc.subcore_barrier()

    def wcp(k, carry):
        q = s + k * NS
        pltpu.sync_copy(acc.at[pl.ds(q * WR, WR)], agg.at[c, pl.ds(q * WR, WR)])
        return carry

    lax.fori_loop(0, (NWCH - s + NS - 1) // NS, wcp, 0)


BPW = B // (NC * NS)


@functools.partial(
    pl.kernel,
    out_type=jax.ShapeDtypeStruct((B, D), jnp.float32),
    mesh=_mesh,
    scratch_types=[
        pltpu.VMEM((BPW,), jnp.int32),
        pltpu.VMEM((BPW, D), jnp.float32),
        pltpu.SemaphoreType.DMA,
    ],
)
def _batch_gather(table, idx, out, idx_v, rows_v, sem):
    wid = lax.axis_index("s") * NC + lax.axis_index("c")
    base = wid * BPW
    pltpu.sync_copy(idx.at[pl.ds(base, BPW)], idx_v)
    pltpu.async_copy(table.at[idx_v], rows_v, sem).wait()
    pltpu.sync_copy(rows_v, out.at[pl.ds(base, BPW)])


TN = 1000


def _mm1_body(a_ref, w_ref, b_ref, o_ref):
    a = a_ref[...]
    x = jnp.concatenate([a[0], a[1]], axis=1)
    y = jnp.dot(x, w_ref[...], preferred_element_type=jnp.float32) + b_ref[...]
    y = jnp.maximum(y, 0.0)
    o_ref[0] = y[:, :H]
    o_ref[1] = y[:, H:]


_mm1 = pl.pallas_call(
    _mm1_body,
    grid=(N_NODES // TN,),
    in_specs=[
        pl.BlockSpec((NC, TN, H), lambda i: (0, i, 0)),
        pl.BlockSpec((D, D), lambda i: (0, 0)),
        pl.BlockSpec((1, D), lambda i: (0, 0)),
    ],
    out_specs=pl.BlockSpec((NC, TN, H), lambda i: (0, i, 0)),
    out_shape=jax.ShapeDtypeStruct((NC, N_NODES, H), jnp.float32),
)


def _mm2_body(a_ref, pa_ref, w_ref, pb_ref, b_ref, o_ref):
    a = a_ref[...]
    x = jnp.concatenate([a[0], a[1]], axis=1)
    y = jnp.dot(x, w_ref[...], preferred_element_type=jnp.float32)
    y = y + jnp.dot(pa_ref[...], pb_ref[...], preferred_element_type=jnp.float32)
    o_ref[...] = y + b_ref[...]


_mm2 = pl.pallas_call(
    _mm2_body,
    grid=(N_NODES // TN,),
    in_specs=[
        pl.BlockSpec((NC, TN, H), lambda i: (0, i, 0)),
        pl.BlockSpec((TN, RANK), lambda i: (i, 0)),
        pl.BlockSpec((D, D), lambda i: (0, 0)),
        pl.BlockSpec((RANK, D), lambda i: (0, 0)),
        pl.BlockSpec((1, D), lambda i: (0, 0)),
    ],
    out_specs=pl.BlockSpec((TN, D), lambda i: (i, 0)),
    out_shape=jax.ShapeDtypeStruct((N_NODES, D), jnp.float32),
)


SB = 1024


def _sel_body(iv_ref, g_ref, base_ref, oov_ref, o_ref):
    m = iv_ref[...] > 0
    o_ref[...] = jnp.where(m, g_ref[...], base_ref[...] + oov_ref[...])


_sel = pl.pallas_call(
    _sel_body,
    grid=(B // SB,),
    in_specs=[
        pl.BlockSpec((SB, 1), lambda i: (i, 0)),
        pl.BlockSpec((SB, D), lambda i: (i, 0)),
        pl.BlockSpec((SB, D), lambda i: (i, 0)),
        pl.BlockSpec((1, D), lambda i: (0, 0)),
    ],
    out_specs=pl.BlockSpec((SB, D), lambda i: (i, 0)),
    out_shape=jax.ShapeDtypeStruct((B, D), jnp.float32),
)


def kernel(base_embedding, node_idx, in_vocab, edge_index, edge_weight, emb,
           W1, b1, W2, b2, pert_A, pert_B, oov_weight):
    src = edge_index[0].astype(jnp.int32)
    dst = edge_index[1].astype(jnp.int32)
    packed = jnp.stack([src, dst], axis=0).reshape(2, NS, NCH, C)
    packed = jnp.transpose(packed, (1, 2, 0, 3))
    off = jnp.array([N_NODES, 0], jnp.int32)[None, None, :, None]
    edges = jnp.stack([packed, packed + off], axis=0)
    ws = edge_weight.reshape(NS, NCH, C)

    emb2 = jnp.concatenate([emb[:, :H], emb[:, H:]], axis=0)

    zrows = jnp.zeros((ZR, H), jnp.float32)

    agg1 = _edge_agg(emb2, edges, ws, zrows)
    h1 = _mm1(agg1, W1, b1.reshape(1, D))
    agg2 = _edge_agg(h1.reshape(NC * N_NODES, H), edges, ws, zrows)
    h2 = _mm2(agg2, pert_A, W2, pert_B, b2.reshape(1, D))
    g = _batch_gather(h2, node_idx.astype(jnp.int32))
    out = _sel(in_vocab.astype(jnp.int32).reshape(B, 1), g,
               base_embedding, oov_weight)
    return out


# speedup vs baseline: 1.9657x; 1.9657x over previous
import functools

import jax
import jax.numpy as jnp
from jax import lax
from jax.experimental import pallas as pl
from jax.experimental.pallas import tpu as pltpu
from jax.experimental.pallas import tpu_sc as plsc

N_NODES = 10000
N_EDGES = 160000
D = 256
H = 128
RANK = 32
B = 4096

NC = 2
NS = 16
L = 16

EPT = N_EDGES // NS
C = 80
NBUF = 3
NCH = 126
EPTP = NCH * C
ZR = 400
NZCH = N_NODES // ZR
WR = 80
NWCH = N_NODES // WR

_mesh = plsc.VectorSubcoreMesh(core_axis_name="c", subcore_axis_name="s")


@functools.partial(
    pl.kernel,
    out_type=jax.ShapeDtypeStruct((NC, N_NODES, H), jnp.float32),
    mesh=_mesh,
    scratch_types=[
        pltpu.VMEM((NBUF, 2, C), jnp.int32),
        pltpu.VMEM((NCH, C), jnp.float32),
        pltpu.VMEM((NBUF, C, H), jnp.float32),
        pltpu.VMEM_SHARED((N_NODES, H), jnp.float32),
        pltpu.SemaphoreType.DMA((NBUF,)),
        pltpu.SemaphoreType.DMA((NBUF,)),
    ],
)
def _edge_agg(x2, edges, ws, zrows, agg, ec_v, w_v, rows_v, acc, gsem, ssem):
    c = lax.axis_index("c")
    s = lax.axis_index("s")

    def zcp(k, carry):
        q = s + k * NS
        pltpu.sync_copy(zrows, acc.at[pl.ds(q * ZR, ZR)])
        return carry

    lax.fori_loop(0, (NZCH - s + NS - 1) // NS, zcp, 0)

    pltpu.sync_copy(ws.at[s], w_v)
    plsc.subcore_barrier()

    def stage(j, b):
        pltpu.sync_copy(edges.at[c, s, j], ec_v.at[b])
        pltpu.async_copy(x2.at[ec_v.at[b, 0]], rows_v.at[b], gsem.at[b])

    def wait_gather(b):
        pltpu.make_async_copy(x2.at[pl.ds(0, C)], rows_v.at[b],
                              gsem.at[b]).wait()

    def wait_scatter(b):
        pltpu.make_async_copy(rows_v.at[b], acc.at[pl.ds(0, C)],
                              ssem.at[b]).wait()

    stage(0, 0)
    stage(1, 1)

    def process(j, b):
        wait_gather(b)

        def scale16(e16, carry2):
            w16 = w_v[j, pl.ds(e16 * L, L)]
            for l in range(L):
                wsp = w16.at[jnp.full((L,), l, jnp.int32)].get(
                    mode="promise_in_bounds")
                e = e16 * L + l
                for k in range(H // L):
                    sl = pl.ds(k * L, L)
                    rows_v[b, e, sl] = rows_v[b, e, sl] * wsp
            return carry2

        lax.fori_loop(0, C // L, scale16, 0)
        pltpu.async_copy(rows_v.at[b], acc.at[ec_v.at[b, 1]], ssem.at[b],
                         add=True)

    def chunk3(i, carry):
        for t in range(NBUF):
            j = NBUF * i + t
            b2 = (t + 2) % NBUF

            @pl.when(j + 2 < NCH)
            def _(j=j, b2=b2):
                @pl.when(j >= 1)
                def _():
                    wait_scatter(b2)
                stage(j + 2, b2)

            process(j, t)
        return carry

    lax.fori_loop(0, NCH // NBUF, chunk3, 0)
    for b in range(NBUF):
        wait_scatter(b)
    plsc.subcore_barrier()

    def wcp(k, carry):
        q = s + k * NS
        pltpu.sync_copy(acc.at[pl.ds(q * WR, WR)], agg.at[c, pl.ds(q * WR, WR)])
        return carry

    lax.fori_loop(0, (NWCH - s + NS - 1) // NS, wcp, 0)


BPW = B // (NC * NS)


@functools.partial(
    pl.kernel,
    out_type=jax.ShapeDtypeStruct((B, D), jnp.float32),
    mesh=_mesh,
    scratch_types=[
        pltpu.VMEM((BPW,), jnp.int32),
        pltpu.VMEM((BPW, D), jnp.float32),
        pltpu.SemaphoreType.DMA,
    ],
)
def _batch_gather(table, idx, out, idx_v, rows_v, sem):
    wid = lax.axis_index("s") * NC + lax.axis_index("c")
    base = wid * BPW
    pltpu.sync_copy(idx.at[pl.ds(base, BPW)], idx_v)
    pltpu.async_copy(table.at[idx_v], rows_v, sem).wait()
    pltpu.sync_copy(rows_v, out.at[pl.ds(base, BPW)])


TN = 1000


def _mm1_body(a_ref, w_ref, b_ref, o_ref):
    a = a_ref[...]
    x = jnp.concatenate([a[0], a[1]], axis=1)
    y = jnp.dot(x, w_ref[...], preferred_element_type=jnp.float32) + b_ref[...]
    y = jnp.maximum(y, 0.0)
    o_ref[0] = y[:, :H]
    o_ref[1] = y[:, H:]


_mm1 = pl.pallas_call(
    _mm1_body,
    grid=(N_NODES // TN,),
    in_specs=[
        pl.BlockSpec((NC, TN, H), lambda i: (0, i, 0)),
        pl.BlockSpec((D, D), lambda i: (0, 0)),
        pl.BlockSpec((1, D), lambda i: (0, 0)),
    ],
    out_specs=pl.BlockSpec((NC, TN, H), lambda i: (0, i, 0)),
    out_shape=jax.ShapeDtypeStruct((NC, N_NODES, H), jnp.float32),
)


def _mm2_body(a_ref, pa_ref, w_ref, pb_ref, b_ref, o_ref):
    a = a_ref[...]
    x = jnp.concatenate([a[0], a[1]], axis=1)
    y = jnp.dot(x, w_ref[...], preferred_element_type=jnp.float32)
    y = y + jnp.dot(pa_ref[...], pb_ref[...], preferred_element_type=jnp.float32)
    o_ref[...] = y + b_ref[...]


_mm2 = pl.pallas_call(
    _mm2_body,
    grid=(N_NODES // TN,),
    in_specs=[
        pl.BlockSpec((NC, TN, H), lambda i: (0, i, 0)),
        pl.BlockSpec((TN, RANK), lambda i: (i, 0)),
        pl.BlockSpec((D, D), lambda i: (0, 0)),
        pl.BlockSpec((RANK, D), lambda i: (0, 0)),
        pl.BlockSpec((1, D), lambda i: (0, 0)),
    ],
    out_specs=pl.BlockSpec((TN, D), lambda i: (i, 0)),
    out_shape=jax.ShapeDtypeStruct((N_NODES, D), jnp.float32),
)


SB = 1024


def _sel_body(iv_ref, g_ref, base_ref, oov_ref, o_ref):
    m = iv_ref[...] > 0
    o_ref[...] = jnp.where(m, g_ref[...], base_ref[...] + oov_ref[...])


_sel = pl.pallas_call(
    _sel_body,
    grid=(B // SB,),
    in_specs=[
        pl.BlockSpec((SB, 1), lambda i: (i, 0)),
        pl.BlockSpec((SB, D), lambda i: (i, 0)),
        pl.BlockSpec((SB, D), lambda i: (i, 0)),
        pl.BlockSpec((1, D), lambda i: (0, 0)),
    ],
    out_specs=pl.BlockSpec((SB, D), lambda i: (i, 0)),
    out_shape=jax.ShapeDtypeStruct((B, D), jnp.float32),
)


def kernel(base_embedding, node_idx, in_vocab, edge_index, edge_weight, emb,
           W1, b1, W2, b2, pert_A, pert_B, oov_weight):
    src = edge_index[0].astype(jnp.int32)
    dst = edge_index[1].astype(jnp.int32)
    pad = EPTP - EPT
    packed = jnp.stack([src, dst], axis=0).reshape(2, NS, EPT)
    packed = jnp.pad(packed, ((0, 0), (0, 0), (0, pad)))
    packed = packed.reshape(2, NS, NCH, C)
    packed = jnp.transpose(packed, (1, 2, 0, 3))
    off = jnp.array([N_NODES, 0], jnp.int32)[None, None, :, None]
    edges = jnp.stack([packed, packed + off], axis=0)
    ws = jnp.pad(edge_weight.reshape(NS, EPT),
                 ((0, 0), (0, pad))).reshape(NS, NCH, C)

    emb2 = jnp.concatenate([emb[:, :H], emb[:, H:]], axis=0)

    zrows = jnp.zeros((ZR, H), jnp.float32)

    agg1 = _edge_agg(emb2, edges, ws, zrows)
    h1 = _mm1(agg1, W1, b1.reshape(1, D))
    agg2 = _edge_agg(h1.reshape(NC * N_NODES, H), edges, ws, zrows)
    h2 = _mm2(agg2, pert_A, W2, pert_B, b2.reshape(1, D))
    g = _batch_gather(h2, node_idx.astype(jnp.int32))
    out = _sel(in_vocab.astype(jnp.int32).reshape(B, 1), g,
               base_embedding, oov_weight)
    return out
